# Initial kernel scaffold; baseline (speedup 1.0000x reference)
import jax
import jax.numpy as jnp
from jax.experimental import pallas as pl


def kernel(pre_w, fine_w):
    raise NotImplementedError("write your pallas kernel here")


# speedup vs baseline: 19.6841x; 19.6841x over previous
import functools

import jax
import jax.numpy as jnp
from jax import lax
from jax.experimental import pallas as pl
from jax.experimental.pallas import tpu as pltpu
from jax.experimental.pallas import tpu_sc as plsc

N_ROWS = 4096
N_COLS = 4096
N_TOTAL = N_ROWS * N_COLS
K_SEL = max(1, int(0.01 * N_TOTAL))

NC = 2
NS = 16
NW = NC * NS
LANES = 16

PER_TILE = N_TOTAL // NW
CHUNK = 16384
N_CHUNKS = PER_TILE // CHUNK
UNROLL = 1

PASSES = (
    dict(W=11, HI=31, LO=20),
    dict(W=11, HI=20, LO=9),
    dict(W=9, HI=9, LO=0),
)

TC_BLOCK_ROWS = 256


def _bits_body(pre_ref, fine_ref, out_ref):
    d = fine_ref[...] - pre_ref[...]
    out_ref[...] = lax.bitcast_convert_type(jnp.abs(d), jnp.int32)


def _compute_bits(pre, fine):
    grid = N_ROWS // TC_BLOCK_ROWS
    return pl.pallas_call(
        _bits_body,
        grid=(grid,),
        in_specs=[pl.BlockSpec((TC_BLOCK_ROWS, N_COLS), lambda i: (i, 0))] * 2,
        out_specs=pl.BlockSpec((TC_BLOCK_ROWS, N_COLS), lambda i: (i, 0)),
        out_shape=jax.ShapeDtypeStruct((N_ROWS, N_COLS), jnp.int32),
    )(pre, fine)


def _make_sc_hist(W, HI, LO):
    nbuckets = 1 << W
    hsize = nbuckets * LANES
    mesh = plsc.VectorSubcoreMesh(core_axis_name="c", subcore_axis_name="s")

    @functools.partial(
        pl.kernel,
        out_type=jax.ShapeDtypeStruct((NW * hsize,), jnp.int32),
        mesh=mesh,
        scratch_types=[
            pltpu.VMEM((hsize,), jnp.int32),
            pltpu.VMEM((128,), jnp.int32),
            pltpu.VMEM((CHUNK,), jnp.int32),
            pltpu.VMEM((CHUNK,), jnp.int32),
            pltpu.SemaphoreType.DMA,
            pltpu.SemaphoreType.DMA,
        ],
        compiler_params=pltpu.CompilerParams(needs_layout_passes=False),
    )
    def hist_kernel(bits_hbm, prev_hbm, out_hbm, hist, pbuf, buf0, buf1,
                    sem0, sem1):
        cid = lax.axis_index("c")
        sid = lax.axis_index("s")
        wid = sid * NC + cid
        base = wid * PER_TILE

        zeros16 = jnp.zeros((LANES,), jnp.int32)
        ones16 = jnp.ones((LANES,), jnp.int32)
        lane = lax.iota(jnp.int32, LANES)

        def zbody(i, _):
            hist[pl.ds(i * LANES, LANES)] = zeros16
            return 0

        lax.fori_loop(0, hsize // LANES, zbody, 0)

        pltpu.sync_copy(prev_hbm.at[1], pbuf)
        pvec = pbuf[pl.ds(0, LANES)]

        bufs = (buf0, buf1)
        sems = (sem0, sem1)
        pending = pltpu.async_copy(bits_hbm.at[pl.ds(base, CHUNK)], buf0, sem0)
        for g in range(N_CHUNKS):
            pending.wait()
            if g + 1 < N_CHUNKS:
                pending = pltpu.async_copy(
                    bits_hbm.at[pl.ds(base + (g + 1) * CHUNK, CHUNK)],
                    bufs[(g + 1) % 2], sems[(g + 1) % 2])
            buf = bufs[g % 2]

            def cbody(i, _, buf=buf):
                off = i * (LANES * UNROLL)
                for u in range(UNROLL):
                    v = buf[pl.ds(off + u * LANES, LANES)]
                    member = (v >> HI) == pvec
                    dig = (v >> LO) & (nbuckets - 1)
                    idx = (dig << 4) + lane
                    plsc.addupdate_scatter(hist, [idx], ones16, mask=member)
                return 0

            lax.fori_loop(0, CHUNK // (LANES * UNROLL), cbody, 0)

        pltpu.sync_copy(hist, out_hbm.at[pl.ds(wid * hsize, hsize)])

    return hist_kernel


def _make_select(W, R):
    def select_body(hist_ref, prev_ref, out_ref):
        h = hist_ref[...]
        C = jnp.sum(jnp.sum(h, axis=3), axis=0).astype(jnp.float32)
        krem = prev_ref[0, 0]
        prefix = prev_ref[1, 0]
        kf = krem.astype(jnp.float32)

        ge = (lax.broadcasted_iota(jnp.int32, (128, 128), 0)
              >= lax.broadcasted_iota(jnp.int32, (128, 128), 1))
        RS = jnp.dot(C, ge.astype(jnp.float32),
                     preferred_element_type=jnp.float32,
                     precision=lax.Precision.HIGHEST)


        g_rows = [None] * R
        acc = jnp.float32(0.0)
        for r in reversed(range(R)):
            g_rows[r] = RS[r:r + 1, :] + acc
            acc = acc + RS[r, 0]

        cnt = jnp.float32(0.0)
        for r in range(R):
            cnt = cnt + jnp.sum((g_rows[r] >= kf).astype(jnp.float32))
        d = cnt - 1.0

        col = lax.broadcasted_iota(jnp.int32, (1, 128), 1).astype(jnp.float32)
        cd = jnp.float32(0.0)
        gd = jnp.float32(0.0)
        for r in range(R):
            seld = ((col + (128.0 * r)) == d).astype(jnp.float32)
            cd = cd + jnp.sum(seld * C[r:r + 1, :])
            gd = gd + jnp.sum(seld * g_rows[r])
        krem_out = kf - (gd - cd)

        prefix_out = (prefix << W) | d.astype(jnp.int32)
        row = lax.broadcasted_iota(jnp.int32, (8, 128), 0)
        out_ref[...] = jnp.where(
            row == 0, krem_out.astype(jnp.int32),
            jnp.where(row == 1, prefix_out, 0))

    return pl.pallas_call(
        select_body,
        out_shape=jax.ShapeDtypeStruct((8, 128), jnp.int32),
    )


def _mask_body(bits_ref, thr_ref, out_ref):
    t = thr_ref[1, 0]
    out_ref[...] = (bits_ref[...] > t).astype(jnp.float32)


def _compute_mask(bits, thr):
    grid = N_ROWS // TC_BLOCK_ROWS
    return pl.pallas_call(
        _mask_body,
        grid=(grid,),
        in_specs=[
            pl.BlockSpec((TC_BLOCK_ROWS, N_COLS), lambda i: (i, 0)),
            pl.BlockSpec((8, 128), lambda i: (0, 0)),
        ],
        out_specs=pl.BlockSpec((TC_BLOCK_ROWS, N_COLS), lambda i: (i, 0)),
        out_shape=jax.ShapeDtypeStruct((N_ROWS, N_COLS), jnp.float32),
    )(bits, thr)


_SC_HISTS = [_make_sc_hist(p["W"], p["HI"], p["LO"]) for p in PASSES]
_SELECTS = [_make_select(p["W"], (1 << p["W"]) // 128) for p in PASSES]


def kernel(pre_w, fine_w):
    bits = _compute_bits(pre_w, fine_w)
    bits1d = bits.reshape(N_TOTAL)

    prev = jnp.zeros((8, 128), jnp.int32).at[0].set(K_SEL)
    for p, spec in enumerate(PASSES):
        R = (1 << spec["W"]) // 128
        hist = _SC_HISTS[p](bits1d, prev)
        histr = hist.reshape(NW, R, 128, LANES)
        prev = _SELECTS[p](histr, prev)

    mask = _compute_mask(bits, prev)
    total_bytes = jnp.asarray(N_TOTAL / 8.0, dtype=jnp.float32)
    return mask, total_bytes


# speedup vs baseline: 21.2822x; 1.0812x over previous
import functools

import jax
import jax.numpy as jnp
from jax import lax
from jax.experimental import pallas as pl
from jax.experimental.pallas import tpu as pltpu
from jax.experimental.pallas import tpu_sc as plsc

N_ROWS = 4096
N_COLS = 4096
N_TOTAL = N_ROWS * N_COLS
K_SEL = max(1, int(0.01 * N_TOTAL))

NC = 2
NS = 16
NW = NC * NS
LANES = 16

PER_TILE = N_TOTAL // NW
CHUNK = 16384
N_CHUNKS = PER_TILE // CHUNK
UNROLL = 8

PASSES = (
    dict(W=11, HI=31, LO=20),
    dict(W=11, HI=20, LO=9),
    dict(W=9, HI=9, LO=0),
)

TC_BLOCK_ROWS = 256


def _bits_body(pre_ref, fine_ref, out_ref):
    d = fine_ref[...] - pre_ref[...]
    out_ref[...] = lax.bitcast_convert_type(jnp.abs(d), jnp.int32)


def _compute_bits(pre, fine):
    grid = N_ROWS // TC_BLOCK_ROWS
    return pl.pallas_call(
        _bits_body,
        grid=(grid,),
        in_specs=[pl.BlockSpec((TC_BLOCK_ROWS, N_COLS), lambda i: (i, 0))] * 2,
        out_specs=pl.BlockSpec((TC_BLOCK_ROWS, N_COLS), lambda i: (i, 0)),
        out_shape=jax.ShapeDtypeStruct((N_ROWS, N_COLS), jnp.int32),
    )(pre, fine)


def _make_sc_hist(W, HI, LO, FIRST):
    nbuckets = 1 << W
    hsize = nbuckets * LANES
    idx_mask = (nbuckets - 1) << 4
    mesh = plsc.VectorSubcoreMesh(core_axis_name="c", subcore_axis_name="s")

    @functools.partial(
        pl.kernel,
        out_type=jax.ShapeDtypeStruct((NW * hsize,), jnp.int32),
        mesh=mesh,
        scratch_types=[
            pltpu.VMEM((hsize,), jnp.int32),
            pltpu.VMEM((128,), jnp.int32),
            pltpu.VMEM((CHUNK,), jnp.int32),
            pltpu.VMEM((CHUNK,), jnp.int32),
            pltpu.SemaphoreType.DMA,
            pltpu.SemaphoreType.DMA,
        ],
        compiler_params=pltpu.CompilerParams(needs_layout_passes=False),
    )
    def hist_kernel(bits_hbm, prev_hbm, out_hbm, hist, pbuf, buf0, buf1,
                    sem0, sem1):
        cid = lax.axis_index("c")
        sid = lax.axis_index("s")
        wid = sid * NC + cid
        base = wid * PER_TILE

        zeros16 = jnp.zeros((LANES,), jnp.int32)
        ones16 = jnp.ones((LANES,), jnp.int32)
        lane = lax.iota(jnp.int32, LANES)

        def zbody(i, _):
            hist[pl.ds(i * LANES, LANES)] = zeros16
            return 0

        lax.fori_loop(0, hsize // LANES, zbody, 0)

        if FIRST:
            pvec = zeros16
        else:
            pltpu.sync_copy(prev_hbm.at[1], pbuf)
            pvec = pbuf[pl.ds(0, LANES)]

        bufs = (buf0, buf1)
        sems = (sem0, sem1)
        pending = pltpu.async_copy(bits_hbm.at[pl.ds(base, CHUNK)], buf0, sem0)
        for g in range(N_CHUNKS):
            pending.wait()
            if g + 1 < N_CHUNKS:
                pending = pltpu.async_copy(
                    bits_hbm.at[pl.ds(base + (g + 1) * CHUNK, CHUNK)],
                    bufs[(g + 1) % 2], sems[(g + 1) % 2])
            buf = bufs[g % 2]

            def cbody(i, _, buf=buf):
                off = i * (LANES * UNROLL)
                for u in range(UNROLL):
                    v = buf[pl.ds(off + u * LANES, LANES)]
                    if LO >= 4:
                        idx = ((v >> (LO - 4)) & idx_mask) | lane
                    else:
                        idx = ((v << (4 - LO)) & idx_mask) | lane
                    if FIRST:
                        plsc.addupdate_scatter(hist, [idx], ones16)
                    else:
                        member = (v >> HI) == pvec
                        plsc.addupdate_scatter(hist, [idx], ones16,
                                               mask=member)
                return 0

            lax.fori_loop(0, CHUNK // (LANES * UNROLL), cbody, 0)

        pltpu.sync_copy(hist, out_hbm.at[pl.ds(wid * hsize, hsize)])

    return hist_kernel


def _make_select(W, R):
    def select_body(hist_ref, prev_ref, out_ref):
        h = hist_ref[...]
        C = jnp.sum(jnp.sum(h, axis=3), axis=0).astype(jnp.float32)
        krem = prev_ref[0, 0]
        prefix = prev_ref[1, 0]
        kf = krem.astype(jnp.float32)

        ge = (lax.broadcasted_iota(jnp.int32, (128, 128), 0)
              >= lax.broadcasted_iota(jnp.int32, (128, 128), 1))
        RS = jnp.dot(C, ge.astype(jnp.float32),
                     preferred_element_type=jnp.float32,
                     precision=lax.Precision.HIGHEST)


        g_rows = [None] * R
        acc = jnp.float32(0.0)
        for r in reversed(range(R)):
            g_rows[r] = RS[r:r + 1, :] + acc
            acc = acc + RS[r, 0]

        cnt = jnp.float32(0.0)
        for r in range(R):
            cnt = cnt + jnp.sum((g_rows[r] >= kf).astype(jnp.float32))
        d = cnt - 1.0

        col = lax.broadcasted_iota(jnp.int32, (1, 128), 1).astype(jnp.float32)
        cd = jnp.float32(0.0)
        gd = jnp.float32(0.0)
        for r in range(R):
            seld = ((col + (128.0 * r)) == d).astype(jnp.float32)
            cd = cd + jnp.sum(seld * C[r:r + 1, :])
            gd = gd + jnp.sum(seld * g_rows[r])
        krem_out = kf - (gd - cd)

        prefix_out = (prefix << W) | d.astype(jnp.int32)
        row = lax.broadcasted_iota(jnp.int32, (8, 128), 0)
        out_ref[...] = jnp.where(
            row == 0, krem_out.astype(jnp.int32),
            jnp.where(row == 1, prefix_out, 0))

    return pl.pallas_call(
        select_body,
        out_shape=jax.ShapeDtypeStruct((8, 128), jnp.int32),
    )


def _mask_body(bits_ref, thr_ref, out_ref):
    t = thr_ref[1, 0]
    out_ref[...] = (bits_ref[...] > t).astype(jnp.float32)


def _compute_mask(bits, thr):
    grid = N_ROWS // TC_BLOCK_ROWS
    return pl.pallas_call(
        _mask_body,
        grid=(grid,),
        in_specs=[
            pl.BlockSpec((TC_BLOCK_ROWS, N_COLS), lambda i: (i, 0)),
            pl.BlockSpec((8, 128), lambda i: (0, 0)),
        ],
        out_specs=pl.BlockSpec((TC_BLOCK_ROWS, N_COLS), lambda i: (i, 0)),
        out_shape=jax.ShapeDtypeStruct((N_ROWS, N_COLS), jnp.float32),
    )(bits, thr)


_SC_HISTS = [_make_sc_hist(p["W"], p["HI"], p["LO"], i == 0)
             for i, p in enumerate(PASSES)]
_SELECTS = [_make_select(p["W"], (1 << p["W"]) // 128) for p in PASSES]


def kernel(pre_w, fine_w):
    bits = _compute_bits(pre_w, fine_w)
    bits1d = bits.reshape(N_TOTAL)

    prev = jnp.zeros((8, 128), jnp.int32).at[0].set(K_SEL)
    for p, spec in enumerate(PASSES):
        R = (1 << spec["W"]) // 128
        hist = _SC_HISTS[p](bits1d, prev)
        histr = hist.reshape(NW, R, 128, LANES)
        prev = _SELECTS[p](histr, prev)

    mask = _compute_mask(bits, prev)
    total_bytes = jnp.asarray(N_TOTAL / 8.0, dtype=jnp.float32)
    return mask, total_bytes


# speedup vs baseline: 26.2732x; 1.2345x over previous
import functools

import jax
import jax.numpy as jnp
from jax import lax
from jax.experimental import pallas as pl
from jax.experimental.pallas import tpu as pltpu
from jax.experimental.pallas import tpu_sc as plsc

N_ROWS = 4096
N_COLS = 4096
N_TOTAL = N_ROWS * N_COLS
K_SEL = max(1, int(0.01 * N_TOTAL))

NC = 2
NS = 16
NW = NC * NS
LANES = 16

PER_TILE = N_TOTAL // NW
CHUNK = 16384
N_CHUNKS = PER_TILE // CHUNK
UNROLL = 8

PASSES = (
    dict(W=11, HI=31, LO=20),
    dict(W=11, HI=20, LO=9),
    dict(W=9, HI=9, LO=0),
)

CAPL = 2048

TC_BLOCK_ROWS = 256


def _bits_body(pre_ref, fine_ref, out_ref):
    d = fine_ref[...] - pre_ref[...]
    out_ref[...] = lax.bitcast_convert_type(jnp.abs(d), jnp.int32)


def _compute_bits(pre, fine):
    grid = N_ROWS // TC_BLOCK_ROWS
    return pl.pallas_call(
        _bits_body,
        grid=(grid,),
        in_specs=[pl.BlockSpec((TC_BLOCK_ROWS, N_COLS), lambda i: (i, 0))] * 2,
        out_specs=pl.BlockSpec((TC_BLOCK_ROWS, N_COLS), lambda i: (i, 0)),
        out_shape=jax.ShapeDtypeStruct((N_ROWS, N_COLS), jnp.int32),
    )(pre, fine)


def _make_sc_hist(W, HI, LO, FIRST, COMPACT=False):
    nbuckets = 1 << W
    hsize = nbuckets * LANES
    idx_mask = (nbuckets - 1) << 4
    mesh = plsc.VectorSubcoreMesh(core_axis_name="c", subcore_axis_name="s")

    out_type = jax.ShapeDtypeStruct((NW * hsize,), jnp.int32)
    scratch = [
        pltpu.VMEM((hsize,), jnp.int32),
        pltpu.VMEM((128,), jnp.int32),
        pltpu.VMEM((CHUNK,), jnp.int32),
        pltpu.VMEM((CHUNK,), jnp.int32),
        pltpu.SemaphoreType.DMA,
        pltpu.SemaphoreType.DMA,
    ]
    if COMPACT:
        out_type = [
            out_type,
            jax.ShapeDtypeStruct((NW * LANES * CAPL,), jnp.int32),
            jax.ShapeDtypeStruct((NW * LANES,), jnp.int32),
        ]
        scratch += [
            pltpu.VMEM((LANES * CAPL,), jnp.int32),
            pltpu.VMEM((LANES,), jnp.int32),
        ]

    @functools.partial(
        pl.kernel,
        out_type=out_type,
        mesh=mesh,
        scratch_types=scratch,
        compiler_params=pltpu.CompilerParams(needs_layout_passes=False),
    )
    def hist_kernel(bits_hbm, prev_hbm, *rest):
        if COMPACT:
            (out_hbm, cmp_hbm, cnt_hbm, hist, pbuf, buf0, buf1,
             sem0, sem1, cstage, cntbuf) = rest
        else:
            out_hbm, hist, pbuf, buf0, buf1, sem0, sem1 = rest
        cid = lax.axis_index("c")
        sid = lax.axis_index("s")
        wid = sid * NC + cid
        base = wid * PER_TILE

        zeros16 = jnp.zeros((LANES,), jnp.int32)
        ones16 = jnp.ones((LANES,), jnp.int32)
        neg16 = jnp.full((LANES,), -1, jnp.int32)
        lane = lax.iota(jnp.int32, LANES)

        def zbody(i, _):
            hist[pl.ds(i * LANES, LANES)] = zeros16
            return 0

        lax.fori_loop(0, hsize // LANES, zbody, 0)

        if COMPACT:
            def sbody(i, _):
                cstage[pl.ds(i * LANES, LANES)] = neg16
                return 0

            lax.fori_loop(0, CAPL, sbody, 0)

        if FIRST:
            pvec = zeros16
        else:
            pltpu.sync_copy(prev_hbm.at[1], pbuf)
            pvec = pbuf[pl.ds(0, LANES)]

        bufs = (buf0, buf1)
        sems = (sem0, sem1)
        basevec = zeros16
        pending = pltpu.async_copy(bits_hbm.at[pl.ds(base, CHUNK)], buf0, sem0)
        for g in range(N_CHUNKS):
            pending.wait()
            if g + 1 < N_CHUNKS:
                pending = pltpu.async_copy(
                    bits_hbm.at[pl.ds(base + (g + 1) * CHUNK, CHUNK)],
                    bufs[(g + 1) % 2], sems[(g + 1) % 2])
            buf = bufs[g % 2]

            def cbody(i, bv, buf=buf):
                off = i * (LANES * UNROLL)
                for u in range(UNROLL):
                    v = buf[pl.ds(off + u * LANES, LANES)]
                    if LO >= 4:
                        idx = ((v >> (LO - 4)) & idx_mask) | lane
                    else:
                        idx = ((v << (4 - LO)) & idx_mask) | lane
                    if FIRST:
                        plsc.addupdate_scatter(hist, [idx], ones16)
                    else:
                        member = (v >> HI) == pvec
                        plsc.addupdate_scatter(hist, [idx], ones16,
                                               mask=member)
                        if COMPACT:
                            wm = member & (bv < CAPL)
                            pos = ((bv & (CAPL - 1)) << 4) | lane
                            plsc.store_scatter(cstage, [pos], v, mask=wm)
                            bv = bv + jnp.where(member, ones16, zeros16)
                return bv

            basevec = lax.fori_loop(0, CHUNK // (LANES * UNROLL), cbody,
                                    basevec)

        pltpu.sync_copy(hist, out_hbm.at[pl.ds(wid * hsize, hsize)])
        if COMPACT:
            cntbuf[pl.ds(0, LANES)] = basevec
            pltpu.sync_copy(
                cstage, cmp_hbm.at[pl.ds(wid * LANES * CAPL, LANES * CAPL)])
            pltpu.sync_copy(cntbuf, cnt_hbm.at[pl.ds(wid * LANES, LANES)])

    return hist_kernel


def _make_sc_hist_compact3(W, HI, LO):
    nbuckets = 1 << W
    hsize = nbuckets * LANES
    idx_mask = (nbuckets - 1) << 4
    mesh = plsc.VectorSubcoreMesh(core_axis_name="c", subcore_axis_name="s")

    @functools.partial(
        pl.kernel,
        out_type=jax.ShapeDtypeStruct((NW * hsize,), jnp.int32),
        mesh=mesh,
        scratch_types=[
            pltpu.VMEM((hsize,), jnp.int32),
            pltpu.VMEM((128,), jnp.int32),
            pltpu.VMEM((LANES * CAPL,), jnp.int32),
            pltpu.SemaphoreType.DMA,
        ],
        compiler_params=pltpu.CompilerParams(needs_layout_passes=False),
    )
    def hist_kernel(cmp_hbm, prev_hbm, out_hbm, hist, pbuf, cbuf, sem):
        cid = lax.axis_index("c")
        sid = lax.axis_index("s")
        wid = sid * NC + cid

        zeros16 = jnp.zeros((LANES,), jnp.int32)
        ones16 = jnp.ones((LANES,), jnp.int32)
        lane = lax.iota(jnp.int32, LANES)

        pending = pltpu.async_copy(
            cmp_hbm.at[pl.ds(wid * LANES * CAPL, LANES * CAPL)], cbuf, sem)

        def zbody(i, _):
            hist[pl.ds(i * LANES, LANES)] = zeros16
            return 0

        lax.fori_loop(0, hsize // LANES, zbody, 0)

        pltpu.sync_copy(prev_hbm.at[1], pbuf)
        pvec = pbuf[pl.ds(0, LANES)]
        pending.wait()

        def cbody(i, _):
            off = i * (LANES * UNROLL)
            for u in range(UNROLL):
                v = cbuf[pl.ds(off + u * LANES, LANES)]
                idx = ((v << 4) & idx_mask) | lane
                member = (v >> HI) == pvec
                plsc.addupdate_scatter(hist, [idx], ones16, mask=member)
            return 0

        lax.fori_loop(0, CAPL // UNROLL, cbody, 0)

        pltpu.sync_copy(hist, out_hbm.at[pl.ds(wid * hsize, hsize)])

    return hist_kernel


def _make_select(W, R, WITH_CNT=False):
    def select_body(hist_ref, *refs):
        if WITH_CNT:
            cnt_ref, prev_ref, out_ref = refs
        else:
            prev_ref, out_ref = refs
        h = hist_ref[...]
        C = jnp.sum(jnp.sum(h, axis=3), axis=0).astype(jnp.float32)
        krem = prev_ref[0, 0]
        prefix = prev_ref[1, 0]
        kf = krem.astype(jnp.float32)

        ge = (lax.broadcasted_iota(jnp.int32, (128, 128), 0)
              >= lax.broadcasted_iota(jnp.int32, (128, 128), 1))
        RS = jnp.dot(C, ge.astype(jnp.float32),
                     preferred_element_type=jnp.float32,
                     precision=lax.Precision.HIGHEST)


        g_rows = [None] * R
        acc = jnp.float32(0.0)
        for r in reversed(range(R)):
            g_rows[r] = RS[r:r + 1, :] + acc
            acc = acc + RS[r, 0]

        cnt = jnp.float32(0.0)
        for r in range(R):
            cnt = cnt + jnp.sum((g_rows[r] >= kf).astype(jnp.float32))
        d = cnt - 1.0

        col = lax.broadcasted_iota(jnp.int32, (1, 128), 1).astype(jnp.float32)
        cd = jnp.float32(0.0)
        gd = jnp.float32(0.0)
        for r in range(R):
            seld = ((col + (128.0 * r)) == d).astype(jnp.float32)
            cd = cd + jnp.sum(seld * C[r:r + 1, :])
            gd = gd + jnp.sum(seld * g_rows[r])
        krem_out = kf - (gd - cd)

        prefix_out = (prefix << W) | d.astype(jnp.int32)
        row = lax.broadcasted_iota(jnp.int32, (8, 128), 0)
        res = jnp.where(
            row == 0, krem_out.astype(jnp.int32),
            jnp.where(row == 1, prefix_out, 0))
        if WITH_CNT:
            maxc = jnp.max(cnt_ref[...])
            res = jnp.where(row == 2, maxc, res)
        out_ref[...] = res

    return pl.pallas_call(
        select_body,
        out_shape=jax.ShapeDtypeStruct((8, 128), jnp.int32),
    )


def _mask_body(bits_ref, thr_ref, out_ref):
    t = thr_ref[1, 0]
    out_ref[...] = (bits_ref[...] > t).astype(jnp.float32)


def _compute_mask(bits, thr):
    grid = N_ROWS // TC_BLOCK_ROWS
    return pl.pallas_call(
        _mask_body,
        grid=(grid,),
        in_specs=[
            pl.BlockSpec((TC_BLOCK_ROWS, N_COLS), lambda i: (i, 0)),
            pl.BlockSpec((8, 128), lambda i: (0, 0)),
        ],
        out_specs=pl.BlockSpec((TC_BLOCK_ROWS, N_COLS), lambda i: (i, 0)),
        out_shape=jax.ShapeDtypeStruct((N_ROWS, N_COLS), jnp.float32),
    )(bits, thr)


_SC_HIST1 = _make_sc_hist(PASSES[0]["W"], PASSES[0]["HI"], PASSES[0]["LO"],
                          True)
_SC_HIST2 = _make_sc_hist(PASSES[1]["W"], PASSES[1]["HI"], PASSES[1]["LO"],
                          False, COMPACT=True)
_SC_HIST3_FULL = _make_sc_hist(PASSES[2]["W"], PASSES[2]["HI"],
                               PASSES[2]["LO"], False)
_SC_HIST3_CMP = _make_sc_hist_compact3(PASSES[2]["W"], PASSES[2]["HI"],
                                       PASSES[2]["LO"])
_SELECT1 = _make_select(PASSES[0]["W"], (1 << PASSES[0]["W"]) // 128)
_SELECT2 = _make_select(PASSES[1]["W"], (1 << PASSES[1]["W"]) // 128,
                        WITH_CNT=True)
_SELECT3 = _make_select(PASSES[2]["W"], (1 << PASSES[2]["W"]) // 128)


def kernel(pre_w, fine_w):
    bits = _compute_bits(pre_w, fine_w)
    bits1d = bits.reshape(N_TOTAL)

    prev = jnp.zeros((8, 128), jnp.int32).at[0].set(K_SEL)

    R1 = (1 << PASSES[0]["W"]) // 128
    hist = _SC_HIST1(bits1d, prev)
    prev = _SELECT1(hist.reshape(NW, R1, 128, LANES), prev)

    R2 = (1 << PASSES[1]["W"]) // 128
    hist, cmp, cnt = _SC_HIST2(bits1d, prev)
    prev = _SELECT2(hist.reshape(NW, R2, 128, LANES),
                    cnt.reshape(NW * LANES // 128, 128), prev)

    R3 = (1 << PASSES[2]["W"]) // 128
    hist = lax.cond(
        prev[2, 0] <= CAPL,
        lambda: _SC_HIST3_CMP(cmp, prev),
        lambda: _SC_HIST3_FULL(bits1d, prev),
    )
    prev = _SELECT3(hist.reshape(NW, R3, 128, LANES), prev)

    mask = _compute_mask(bits, prev)
    total_bytes = jnp.asarray(N_TOTAL / 8.0, dtype=jnp.float32)
    return mask, total_bytes
